# baseline (speedup 1.0000x reference)
import functools

import jax
import jax.numpy as jnp
from jax import lax
from jax.experimental import pallas as pl
from jax.experimental.pallas import tpu as pltpu
from jax.experimental.pallas import tpu_sc as plsc

_NC = 2
_NS = 16
_NW = _NC * _NS


@functools.lru_cache(maxsize=None)
def _build_gather(B, D, chunk):
    b_per_w = B // _NW
    n_ch = b_per_w // chunk
    mesh = plsc.VectorSubcoreMesh(core_axis_name="c", subcore_axis_name="s")

    @functools.partial(
        pl.kernel,
        mesh=mesh,
        out_type=jax.ShapeDtypeStruct((B, D), jnp.float32),
        compiler_params=pltpu.CompilerParams(use_tc_tiling_on_sc=False),
        scratch_types=[
            pltpu.VMEM((b_per_w,), jnp.int32),
            pltpu.VMEM((2, chunk, D), jnp.float32),
            pltpu.SemaphoreType.DMA,
            pltpu.SemaphoreType.DMA,
            pltpu.SemaphoreType.DMA,
            pltpu.SemaphoreType.DMA,
        ],
    )
    def gather_kernel(ids_hbm, table_hbm, out_hbm, idx_v, rows_v, g0, g1, s0, s1):
        wid = lax.axis_index("s") * _NC + lax.axis_index("c")
        base = wid * b_per_w
        pltpu.sync_copy(ids_hbm.at[pl.ds(base, b_per_w)], idx_v)

        gsem = [g0, g1]
        ssem = [s0, s1]
        gathers = [None, None]
        stores = [None, None]

        def start_gather(c):
            slot = c & 1
            gathers[slot] = pltpu.async_copy(
                table_hbm.at[idx_v.at[pl.ds(c * chunk, chunk)]],
                rows_v.at[slot],
                gsem[slot],
            )

        start_gather(0)
        for c in range(n_ch):
            slot = c & 1
            if c + 1 < n_ch:
                nslot = (c + 1) & 1
                if stores[nslot] is not None:
                    stores[nslot].wait()
                    stores[nslot] = None
                start_gather(c + 1)
            gathers[slot].wait()
            stores[slot] = pltpu.async_copy(
                rows_v.at[slot],
                out_hbm.at[pl.ds(base + c * chunk, chunk)],
                ssem[slot],
            )
        for slot in range(2):
            if stores[slot] is not None:
                stores[slot].wait()

    return gather_kernel


def kernel(input_ids, weight):
    batch, hist = input_ids.shape
    B = batch * hist
    D = weight.shape[1]
    flat_ids = input_ids.reshape(B).astype(jnp.int32)
    out = _build_gather(B, D, 640)(flat_ids, weight)
    return out.reshape(batch, hist, D)
